# baseline (speedup 1.0000x reference)
import functools

import jax
import jax.numpy as jnp
from jax import lax
from jax.experimental import pallas as pl
from jax.experimental.pallas import tpu as pltpu
from jax.experimental.pallas import tpu_sc as plsc

N = 8192
NB = 16
SPLIT = 5120
NWORK = 32
SC_ROWS = N - SPLIT
ROWS_PER_W = SC_ROWS // NWORK
L = 16
GROUPS = ROWS_PER_W // L

TR = 256
CC = 256
NI = SPLIT // TR
NC = N // CC

_LOG1P_C = (1.144709767686436e-05, 0.9991664290428162, -0.4896990954875946,
            0.2838231921195984, -0.1299571990966797, 0.029808765277266502)


def _log1p_poly(u):
    acc = jnp.full_like(u, _LOG1P_C[-1])
    for c in _LOG1P_C[-2::-1]:
        acc = acc * u + c
    return acc


@functools.lru_cache(maxsize=None)
def _get_mesh():
    return plsc.VectorSubcoreMesh(core_axis_name="c", subcore_axis_name="s",
                                  num_cores=2, num_subcores=16)


def _pair_losses(p_r, t_r, pj, tj):
    d = p_r - pj
    dc = jnp.minimum(jnp.maximum(d, -100.0), 100.0)
    u = jnp.exp(jnp.minimum(dc, -dc))
    sp = jnp.maximum(dc, 0.0) + _log1p_poly(u)
    sg = jnp.sign(t_r - tj)
    h = 0.5 * dc
    return sp - h * sg - h


def _sc_pairs_body(p_hbm, t_hbm, b_hbm, part_hbm, cnt_hbm,
                   p_v, t_v, b_v, acc_v, part_v, cnt_v, ends_v):
    c = lax.axis_index("c")
    s = lax.axis_index("s")
    wid = s * 2 + c
    pltpu.sync_copy(p_hbm, p_v.at[pl.ds(0, N)])
    pltpu.sync_copy(t_hbm, t_v.at[pl.ds(0, N)])
    pltpu.sync_copy(b_hbm, b_v)

    zeros = jnp.zeros((L,), jnp.float32)
    p_v[pl.ds(N, L)] = zeros
    t_v[pl.ds(N, L)] = zeros
    for k in range(NB):
        acc_v[pl.ds(k * L, L)] = zeros

    iota = lax.iota(jnp.int32, L)

    def bs_body(_, lohi):
        lo, hi = lohi
        mid = (lo + hi) >> 1
        vals = plsc.load_gather(b_v, [mid])
        le = vals <= iota
        return (jnp.where(le, mid + 1, lo), jnp.where(le, hi, mid))

    ends, _ = lax.fori_loop(0, 13, bs_body,
                            (jnp.zeros((L,), jnp.int32),
                             jnp.full((L,), N, jnp.int32)))
    ends_v[...] = ends
    prev = plsc.load_gather(ends_v, [jnp.maximum(iota - 1, 0)])
    cnt_v[...] = ends - jnp.where(iota == 0, 0, prev)

    def group_body(g, _):
        base_r = SPLIT + (g * L) * NWORK + wid
        r_vec = base_r + NWORK * iota
        b_rs = plsc.load_gather(b_v, [r_vec])
        p_rs = plsc.load_gather(p_v, [r_vec])
        t_rs = plsc.load_gather(t_v, [r_vec])
        e_s = plsc.load_gather(ends_v, [b_rs])

        for l in range(L):
            r = base_r + NWORK * l
            p_r = p_rs[l]
            t_r = t_rs[l]
            slot = b_rs[l] * L
            e = e_s[l]
            ch_lo = (r + 1) >> 4
            ch_hi = (e + 15) >> 4
            nit = (ch_hi - ch_lo + 1) >> 1

            def qbody(q, carry, r=r, e=e, p_r=p_r, t_r=t_r, ch_lo=ch_lo):
                acc0, acc1 = carry
                base0 = (ch_lo + 2 * q) * L
                base1 = base0 + L
                pj0 = p_v[pl.ds(base0, L)]
                tj0 = t_v[pl.ds(base0, L)]
                pj1 = p_v[pl.ds(base1, L)]
                tj1 = t_v[pl.ds(base1, L)]
                idx0 = base0 + iota
                idx1 = base1 + iota
                m0 = (idx0 > r) & (idx0 < e)
                m1 = (idx1 > r) & (idx1 < e)
                l0 = _pair_losses(p_r, t_r, pj0, tj0)
                l1 = _pair_losses(p_r, t_r, pj1, tj1)
                return (acc0 + jnp.where(m0, l0, 0.0),
                        acc1 + jnp.where(m1, l1, 0.0))

            a0 = acc_v[pl.ds(slot, L)]
            a0, a1 = lax.fori_loop(0, nit, qbody, (a0, zeros))
            acc_v[pl.ds(slot, L)] = a0 + a1
        return 0

    lax.fori_loop(0, GROUPS, group_body, 0)

    part_vec = zeros
    for l in range(L):
        part_vec = part_vec + plsc.load_gather(acc_v, [iota * L + l])
    part_v[...] = part_vec
    pltpu.sync_copy(part_v, part_hbm.at[wid])

    @pl.when(wid == 0)
    def _():
        pltpu.sync_copy(cnt_v, cnt_hbm)


def _tc_body(p_c_ref, t_c_ref, b_c_ref, out_ref):
    i = pl.program_id(0)

    @pl.when(i == 0)
    def _init():
        out_ref[...] = jnp.zeros_like(out_ref)

    b_rl = b_c_ref[pl.ds(i, 1), :]
    b_r = jnp.transpose(b_rl)
    p_r = jnp.transpose(p_c_ref[pl.ds(i, 1), :])
    t_r = jnp.transpose(t_c_ref[pl.ds(i, 1), :])
    bmax_r = b_rl[0, TR - 1]
    ce = jnp.sum((b_c_ref[...] <= bmax_r).astype(jnp.int32))
    nchunks = (ce + CC - 1) // CC

    r0 = i * TR
    iota_r = jax.lax.broadcasted_iota(jnp.int32, (TR, 1), 0) + r0

    def one_chunk(j, live):
        jr = jnp.minimum(j, NC - 1)
        c0 = j * CC
        p_c = p_c_ref[pl.ds(jr, 1), :]
        t_c = t_c_ref[pl.ds(jr, 1), :]
        b_c = b_c_ref[pl.ds(jr, 1), :]
        iota_c = jax.lax.broadcasted_iota(jnp.int32, (1, CC), 1) + c0
        mask = (b_r == b_c) & (iota_r < iota_c) & live
        d = p_r - p_c
        dc = jnp.minimum(jnp.maximum(d, -100.0), 100.0)
        u = jnp.exp(jnp.minimum(dc, -dc))
        sp = jnp.maximum(dc, 0.0) + jnp.log(1.0 + u)
        sg = jnp.sign(t_r - t_c)
        h = 0.5 * dc
        loss = sp - h * sg - h
        return jnp.sum(jnp.where(mask, loss, 0.0), axis=1, keepdims=True)

    def chunk2(q, carry):
        acc0, acc1 = carry
        j0 = i + 2 * q
        j1 = j0 + 1
        acc0 = acc0 + one_chunk(j0, True)
        acc1 = acc1 + one_chunk(j1, j1 < nchunks)
        return (acc0, acc1)

    nit = (nchunks - i + 1) >> 1
    z = jnp.zeros((TR, 1), jnp.float32)
    acc0, acc1 = jax.lax.fori_loop(0, nit, chunk2, (z, z))
    acc_rows = acc0 + acc1

    bins = jax.lax.broadcasted_iota(jnp.int32, (1, NB), 1)
    onehot = (b_r == bins)
    out_ref[...] += jnp.sum(jnp.where(onehot, acc_rows, 0.0), axis=0,
                            keepdims=True)


def _tc_final_body(part_ref, tc_ref, cnt_ref, out_ref):
    sums = jnp.sum(part_ref[...], axis=0, keepdims=True) + tc_ref[...]
    nb = cnt_ref[...]
    npairs = (nb * (nb - 1)) >> 1
    safe = jnp.where(npairs > 0, npairs, 1).astype(jnp.float32)
    loss_b = jnp.where(nb >= 2, sums / safe, 0.0)
    total = jnp.sum(loss_b, axis=1, keepdims=True)
    cnt2 = jnp.sum((nb >= 2).astype(jnp.int32), axis=1, keepdims=True)
    out_ref[...] = jnp.where(
        cnt2 > 0, total / jnp.maximum(cnt2, 1).astype(jnp.float32),
        jnp.float32(0.0))


def kernel(pred_scores, true_scores, batch_ids):
    b = batch_ids.astype(jnp.int32)
    part, cnt = pl.kernel(
        _sc_pairs_body,
        out_type=(jax.ShapeDtypeStruct((NWORK, L), jnp.float32),
                  jax.ShapeDtypeStruct((NB,), jnp.int32)),
        mesh=_get_mesh(),
        compiler_params=pltpu.CompilerParams(needs_layout_passes=False),
        scratch_types=(pltpu.VMEM((N + L,), jnp.float32),
                       pltpu.VMEM((N + L,), jnp.float32),
                       pltpu.VMEM((N,), jnp.int32),
                       pltpu.VMEM((NB * L,), jnp.float32),
                       pltpu.VMEM((L,), jnp.float32),
                       pltpu.VMEM((NB,), jnp.int32),
                       pltpu.VMEM((NB,), jnp.int32)),
    )(pred_scores, true_scores, b)
    tc_part = pl.pallas_call(
        _tc_body,
        grid=(NI,),
        in_specs=[
            pl.BlockSpec((NC, CC), lambda i: (0, 0)),
            pl.BlockSpec((NC, CC), lambda i: (0, 0)),
            pl.BlockSpec((NC, CC), lambda i: (0, 0)),
        ],
        out_specs=pl.BlockSpec((1, NB), lambda i: (0, 0)),
        out_shape=jax.ShapeDtypeStruct((1, NB), jnp.float32),
    )(
        pred_scores.reshape(NC, CC), true_scores.reshape(NC, CC),
        b.reshape(NC, CC),
    )
    out = pl.pallas_call(
        _tc_final_body,
        out_shape=jax.ShapeDtypeStruct((1, 1), jnp.float32),
    )(part, tc_part, cnt.reshape(1, NB))
    return out[0, 0]


# speedup vs baseline: 1.0809x; 1.0809x over previous
import functools

import jax
import jax.numpy as jnp
from jax import lax
from jax.experimental import pallas as pl
from jax.experimental.pallas import tpu as pltpu
from jax.experimental.pallas import tpu_sc as plsc

N = 8192
NB = 16
SPLIT = 5632
NWORK = 32
SC_ROWS = N - SPLIT
ROWS_PER_W = SC_ROWS // NWORK
L = 16
GROUPS = ROWS_PER_W // L

TR = 256
CC = 256
NI = SPLIT // TR
NC = N // CC

_LOG1P_C = (1.144709767686436e-05, 0.9991664290428162, -0.4896990954875946,
            0.2838231921195984, -0.1299571990966797, 0.029808765277266502)


def _log1p_poly(u):
    acc = jnp.full_like(u, _LOG1P_C[-1])
    for c in _LOG1P_C[-2::-1]:
        acc = acc * u + c
    return acc


@functools.lru_cache(maxsize=None)
def _get_mesh():
    return plsc.VectorSubcoreMesh(core_axis_name="c", subcore_axis_name="s",
                                  num_cores=2, num_subcores=16)


def _pair_losses(p_r, t_r, pj, tj):
    d = p_r - pj
    dc = jnp.minimum(jnp.maximum(d, -100.0), 100.0)
    u = jnp.exp(jnp.minimum(dc, -dc))
    sp = jnp.maximum(dc, 0.0) + _log1p_poly(u)
    sg = jnp.sign(t_r - tj)
    h = 0.5 * dc
    return sp - h * sg - h


def _sc_pairs_body(p_hbm, t_hbm, b_hbm, part_hbm, cnt_hbm,
                   p_v, t_v, b_v, acc_v, part_v, cnt_v, ends_v):
    c = lax.axis_index("c")
    s = lax.axis_index("s")
    wid = s * 2 + c
    pltpu.sync_copy(p_hbm, p_v.at[pl.ds(0, N)])
    pltpu.sync_copy(t_hbm, t_v.at[pl.ds(0, N)])
    pltpu.sync_copy(b_hbm, b_v)

    zeros = jnp.zeros((L,), jnp.float32)
    p_v[pl.ds(N, L)] = zeros
    t_v[pl.ds(N, L)] = zeros
    for k in range(NB):
        acc_v[pl.ds(k * L, L)] = zeros

    iota = lax.iota(jnp.int32, L)

    def bs_body(_, lohi):
        lo, hi = lohi
        mid = (lo + hi) >> 1
        vals = plsc.load_gather(b_v, [mid])
        le = vals <= iota
        return (jnp.where(le, mid + 1, lo), jnp.where(le, hi, mid))

    ends, _ = lax.fori_loop(0, 13, bs_body,
                            (jnp.zeros((L,), jnp.int32),
                             jnp.full((L,), N, jnp.int32)))
    ends_v[...] = ends
    prev = plsc.load_gather(ends_v, [jnp.maximum(iota - 1, 0)])
    cnt_v[...] = ends - jnp.where(iota == 0, 0, prev)

    def group_body(g, _):
        base_r = SPLIT + (g * L) * NWORK + wid
        r_vec = base_r + NWORK * iota
        b_rs = plsc.load_gather(b_v, [r_vec])
        p_rs = plsc.load_gather(p_v, [r_vec])
        t_rs = plsc.load_gather(t_v, [r_vec])
        e_s = plsc.load_gather(ends_v, [b_rs])

        for l in range(L):
            r = base_r + NWORK * l
            p_r = p_rs[l]
            t_r = t_rs[l]
            slot = b_rs[l] * L
            e = e_s[l]
            ch_lo = (r + 1) >> 4
            ch_hi = (e + 15) >> 4
            nit = (ch_hi - ch_lo + 1) >> 1

            def qbody(q, carry, r=r, e=e, p_r=p_r, t_r=t_r, ch_lo=ch_lo):
                acc0, acc1 = carry
                base0 = (ch_lo + 2 * q) * L
                base1 = base0 + L
                pj0 = p_v[pl.ds(base0, L)]
                tj0 = t_v[pl.ds(base0, L)]
                pj1 = p_v[pl.ds(base1, L)]
                tj1 = t_v[pl.ds(base1, L)]
                idx0 = base0 + iota
                idx1 = base1 + iota
                m0 = (idx0 > r) & (idx0 < e)
                m1 = (idx1 > r) & (idx1 < e)
                l0 = _pair_losses(p_r, t_r, pj0, tj0)
                l1 = _pair_losses(p_r, t_r, pj1, tj1)
                return (acc0 + jnp.where(m0, l0, 0.0),
                        acc1 + jnp.where(m1, l1, 0.0))

            a0 = acc_v[pl.ds(slot, L)]
            a0, a1 = lax.fori_loop(0, nit, qbody, (a0, zeros))
            acc_v[pl.ds(slot, L)] = a0 + a1
        return 0

    lax.fori_loop(0, GROUPS, group_body, 0)

    part_vec = zeros
    for l in range(L):
        part_vec = part_vec + plsc.load_gather(acc_v, [iota * L + l])
    part_v[...] = part_vec
    pltpu.sync_copy(part_v, part_hbm.at[wid])

    @pl.when(wid == 0)
    def _():
        pltpu.sync_copy(cnt_v, cnt_hbm)


def _tc_body(p_c_ref, t_c_ref, b_c_ref, out_ref):
    i = pl.program_id(0)

    @pl.when(i == 0)
    def _init():
        out_ref[...] = jnp.zeros_like(out_ref)

    b_rl = b_c_ref[pl.ds(i, 1), :]
    b_r = jnp.transpose(b_rl)
    p_r = jnp.transpose(p_c_ref[pl.ds(i, 1), :])
    t_r = jnp.transpose(t_c_ref[pl.ds(i, 1), :])
    bmax_r = b_rl[0, TR - 1]
    ce = jnp.sum((b_c_ref[...] <= bmax_r).astype(jnp.int32))
    nchunks = (ce + CC - 1) // CC

    r0 = i * TR
    iota_r = jax.lax.broadcasted_iota(jnp.int32, (TR, 1), 0) + r0

    def one_chunk(j, live):
        jr = jnp.minimum(j, NC - 1)
        c0 = j * CC
        p_c = p_c_ref[pl.ds(jr, 1), :]
        t_c = t_c_ref[pl.ds(jr, 1), :]
        b_c = b_c_ref[pl.ds(jr, 1), :]
        iota_c = jax.lax.broadcasted_iota(jnp.int32, (1, CC), 1) + c0
        mask = (b_r == b_c) & (iota_r < iota_c) & live
        d = p_r - p_c
        dc = jnp.minimum(jnp.maximum(d, -100.0), 100.0)
        u = jnp.exp(jnp.minimum(dc, -dc))
        sp = jnp.maximum(dc, 0.0) + jnp.log(1.0 + u)
        sg = jnp.sign(t_r - t_c)
        h = 0.5 * dc
        loss = sp - h * sg - h
        return jnp.sum(jnp.where(mask, loss, 0.0), axis=1, keepdims=True)

    def chunk2(q, carry):
        acc0, acc1 = carry
        j0 = i + 2 * q
        j1 = j0 + 1
        acc0 = acc0 + one_chunk(j0, True)
        acc1 = acc1 + one_chunk(j1, j1 < nchunks)
        return (acc0, acc1)

    nit = (nchunks - i + 1) >> 1
    z = jnp.zeros((TR, 1), jnp.float32)
    acc0, acc1 = jax.lax.fori_loop(0, nit, chunk2, (z, z))
    acc_rows = acc0 + acc1

    bins = jax.lax.broadcasted_iota(jnp.int32, (1, NB), 1)
    onehot = (b_r == bins)
    out_ref[...] += jnp.sum(jnp.where(onehot, acc_rows, 0.0), axis=0,
                            keepdims=True)


def _tc_final_body(part_ref, tc_ref, cnt_ref, out_ref):
    sums = jnp.sum(part_ref[...], axis=0, keepdims=True) + tc_ref[...]
    nb = cnt_ref[...]
    npairs = (nb * (nb - 1)) >> 1
    safe = jnp.where(npairs > 0, npairs, 1).astype(jnp.float32)
    loss_b = jnp.where(nb >= 2, sums / safe, 0.0)
    total = jnp.sum(loss_b, axis=1, keepdims=True)
    cnt2 = jnp.sum((nb >= 2).astype(jnp.int32), axis=1, keepdims=True)
    out_ref[...] = jnp.where(
        cnt2 > 0, total / jnp.maximum(cnt2, 1).astype(jnp.float32),
        jnp.float32(0.0))


def kernel(pred_scores, true_scores, batch_ids):
    b = batch_ids.astype(jnp.int32)
    part, cnt = pl.kernel(
        _sc_pairs_body,
        out_type=(jax.ShapeDtypeStruct((NWORK, L), jnp.float32),
                  jax.ShapeDtypeStruct((NB,), jnp.int32)),
        mesh=_get_mesh(),
        compiler_params=pltpu.CompilerParams(needs_layout_passes=False),
        scratch_types=(pltpu.VMEM((N + L,), jnp.float32),
                       pltpu.VMEM((N + L,), jnp.float32),
                       pltpu.VMEM((N,), jnp.int32),
                       pltpu.VMEM((NB * L,), jnp.float32),
                       pltpu.VMEM((L,), jnp.float32),
                       pltpu.VMEM((NB,), jnp.int32),
                       pltpu.VMEM((NB,), jnp.int32)),
    )(pred_scores, true_scores, b)
    tc_part = pl.pallas_call(
        _tc_body,
        grid=(NI,),
        in_specs=[
            pl.BlockSpec((NC, CC), lambda i: (0, 0)),
            pl.BlockSpec((NC, CC), lambda i: (0, 0)),
            pl.BlockSpec((NC, CC), lambda i: (0, 0)),
        ],
        out_specs=pl.BlockSpec((1, NB), lambda i: (0, 0)),
        out_shape=jax.ShapeDtypeStruct((1, NB), jnp.float32),
    )(
        pred_scores.reshape(NC, CC), true_scores.reshape(NC, CC),
        b.reshape(NC, CC),
    )
    out = pl.pallas_call(
        _tc_final_body,
        out_shape=jax.ShapeDtypeStruct((1, 1), jnp.float32),
    )(part, tc_part, cnt.reshape(1, NB))
    return out[0, 0]
